# baseline (speedup 1.0000x reference)
import jax
import jax.numpy as jnp
from jax import lax
from jax.experimental import pallas as pl
from jax.experimental.pallas import tpu as pltpu
import jax.experimental.pallas.tpu_sc as plsc

BATCH = 16384
EMBED = 32
PACK = 4
NC = 2
NS = 16
NW = NC * NS
B_PER_W = BATCH // NW
CH = 128


def _gather_body(uidx_hbm, iidx_hbm, ut_hbm, it_hbm, gu_hbm, gi_hbm,
                 idx_v, bidx_v, g_v, sem):
    w = lax.axis_index("s") * NC + lax.axis_index("c")
    base = pl.multiple_of(w * B_PER_W, B_PER_W)

    for idx_hbm, t_hbm, o_hbm in ((uidx_hbm, ut_hbm, gu_hbm),
                                  (iidx_hbm, it_hbm, gi_hbm)):
        pltpu.sync_copy(idx_hbm.at[pl.ds(base, B_PER_W)], idx_v)
        for g in range(B_PER_W // 16):
            iv = idx_v[pl.ds(g * 16, 16)]
            bidx_v[pl.ds(g * 16, 16)] = lax.shift_right_logical(iv, 2)
        copies = []
        for c in range(B_PER_W // CH):
            copies.append(pltpu.async_copy(
                t_hbm.at[bidx_v.at[pl.ds(c * CH, CH)]],
                g_v.at[pl.ds(c * CH, CH)], sem))
        for cp in copies:
            cp.wait()
        pltpu.sync_copy(g_v, o_hbm.at[pl.ds(base, B_PER_W)])


def _sc_gather(user_indices, item_indices, ut, it):
    mesh = plsc.VectorSubcoreMesh(core_axis_name="c", subcore_axis_name="s")
    return pl.kernel(
        _gather_body,
        out_type=(
            jax.ShapeDtypeStruct((BATCH, 128), jnp.float32),
            jax.ShapeDtypeStruct((BATCH, 128), jnp.float32),
        ),
        mesh=mesh,
        scratch_types=[
            pltpu.VMEM((B_PER_W,), jnp.int32),
            pltpu.VMEM((B_PER_W,), jnp.int32),
            pltpu.VMEM((B_PER_W, 128), jnp.float32),
            pltpu.SemaphoreType.DMA,
        ],
        compiler_params=pltpu.CompilerParams(needs_layout_passes=False),
    )(user_indices, item_indices, ut, it)


def _mlp_body(gu_ref, gi_ref, idxu_ref, idxi_ref, w1u_ref, w1i_ref, b1_ref,
              w2_ref, b2_ref, w3_ref, b3_ref, wo_ref, bo_ref, out_ref):
    subu = lax.bitwise_and(idxu_ref[...], 3)
    subi = lax.bitwise_and(idxi_ref[...], 3)
    uv = jnp.zeros_like(gu_ref[:, :EMBED])
    iv = jnp.zeros_like(uv)
    for t in range(PACK):
        uv = uv + jnp.where(subu == t, gu_ref[:, t * EMBED:(t + 1) * EMBED], 0.0)
        iv = iv + jnp.where(subi == t, gi_ref[:, t * EMBED:(t + 1) * EMBED], 0.0)
    h = jnp.dot(uv, w1u_ref[...], preferred_element_type=jnp.float32)
    h = h + jnp.dot(iv, w1i_ref[...], preferred_element_type=jnp.float32)
    h = jnp.maximum(h + b1_ref[...], 0.0)
    h = jnp.maximum(
        jnp.dot(h, w2_ref[...], preferred_element_type=jnp.float32) + b2_ref[...], 0.0)
    h = jnp.maximum(
        jnp.dot(h, w3_ref[...], preferred_element_type=jnp.float32) + b3_ref[...], 0.0)
    o = jnp.dot(h, wo_ref[...], preferred_element_type=jnp.float32) + bo_ref[...]
    out_ref[...] = jax.nn.sigmoid(o)


def _tc_mlp(gu, gi, idxu2, idxi2, W1, b1, W2, b2, W3, b3, Wo, bo):
    BB = 2048
    grid = (BATCH // BB,)
    w1u = W1[:EMBED]
    w1i = W1[EMBED:]
    full = lambda i: (0, 0)
    return pl.pallas_call(
        _mlp_body,
        grid=grid,
        in_specs=[
            pl.BlockSpec((BB, 128), lambda i: (i, 0)),
            pl.BlockSpec((BB, 128), lambda i: (i, 0)),
            pl.BlockSpec((BB, 1), lambda i: (i, 0)),
            pl.BlockSpec((BB, 1), lambda i: (i, 0)),
            pl.BlockSpec((EMBED, 128), full),
            pl.BlockSpec((EMBED, 128), full),
            pl.BlockSpec((1, 128), full),
            pl.BlockSpec((128, 64), full),
            pl.BlockSpec((1, 64), full),
            pl.BlockSpec((64, 32), full),
            pl.BlockSpec((1, 32), full),
            pl.BlockSpec((32, 1), full),
            pl.BlockSpec((1, 1), full),
        ],
        out_specs=pl.BlockSpec((BB, 1), lambda i: (i, 0)),
        out_shape=jax.ShapeDtypeStruct((BATCH, 1), jnp.float32),
    )(gu, gi, idxu2, idxi2, w1u, w1i, b1.reshape(1, 128), W2, b2.reshape(1, 64),
      W3, b3.reshape(1, 32), Wo, bo.reshape(1, 1))


def kernel(user_indices, item_indices, user_table, item_table,
           W1, b1, W2, b2, W3, b3, Wo, bo):
    uidx = user_indices.astype(jnp.int32)
    iidx = item_indices.astype(jnp.int32)
    ut = user_table.reshape(user_table.shape[0] // PACK, EMBED * PACK)
    it = item_table.reshape(item_table.shape[0] // PACK, EMBED * PACK)
    gu, gi = _sc_gather(uidx, iidx, ut, it)
    return _tc_mlp(gu, gi, uidx.reshape(BATCH, 1), iidx.reshape(BATCH, 1),
                   W1, b1, W2, b2, W3, b3, Wo, bo)


# speedup vs baseline: 1.0487x; 1.0487x over previous
import jax
import jax.numpy as jnp
from jax import lax
from jax.experimental import pallas as pl
from jax.experimental.pallas import tpu as pltpu
import jax.experimental.pallas.tpu_sc as plsc

BATCH = 16384
EMBED = 32
NROWS = 1000000
PACK = 4
C = 250368
RB = 512
NBLK = C // RB
LAST_BLK = (NROWS + RB - 1) // RB - 1
NC = 2
NS = 16
NW = NC * NS
B_PER_W = BATCH // NW
CH = 128


def _repack_body(x0_ref, x1_ref, x2_ref, x3_ref, out_ref):
    panes = [jnp.transpose(r[...]) for r in (x0_ref, x1_ref, x2_ref, x3_ref)]
    out_ref[...] = jnp.concatenate(panes, axis=1)


def _tc_repack(tabT):
    in_specs = [
        pl.BlockSpec((EMBED, RB),
                     (lambda i, t=t: (0, jnp.minimum(t * NBLK + i, LAST_BLK))))
        for t in range(PACK)
    ]
    return pl.pallas_call(
        _repack_body,
        grid=(NBLK,),
        in_specs=in_specs,
        out_specs=pl.BlockSpec((RB, EMBED * PACK), lambda i: (i, 0)),
        out_shape=jax.ShapeDtypeStruct((C, EMBED * PACK), jnp.float32),
    )(tabT, tabT, tabT, tabT)


def _pane_id(iv):
    t = (iv >= C).astype(jnp.int32)
    t = t + (iv >= 2 * C).astype(jnp.int32)
    t = t + (iv >= 3 * C).astype(jnp.int32)
    return t


def _gather_body(uidx_hbm, iidx_hbm, ut_hbm, it_hbm, gu_hbm, gi_hbm,
                 idx_v, bidx_v, g_v, sem):
    w = lax.axis_index("s") * NC + lax.axis_index("c")
    base = pl.multiple_of(w * B_PER_W, B_PER_W)

    for idx_hbm, t_hbm, o_hbm in ((uidx_hbm, ut_hbm, gu_hbm),
                                  (iidx_hbm, it_hbm, gi_hbm)):
        pltpu.sync_copy(idx_hbm.at[pl.ds(base, B_PER_W)], idx_v)
        for g in range(B_PER_W // 16):
            iv = idx_v[pl.ds(g * 16, 16)]
            bidx_v[pl.ds(g * 16, 16)] = iv - _pane_id(iv) * C
        copies = []
        for c in range(B_PER_W // CH):
            copies.append(pltpu.async_copy(
                t_hbm.at[bidx_v.at[pl.ds(c * CH, CH)]],
                g_v.at[pl.ds(c * CH, CH)], sem))
        for cp in copies:
            cp.wait()
        pltpu.sync_copy(g_v, o_hbm.at[pl.ds(base, B_PER_W)])


def _sc_gather(user_indices, item_indices, ut, it):
    mesh = plsc.VectorSubcoreMesh(core_axis_name="c", subcore_axis_name="s")
    return pl.kernel(
        _gather_body,
        out_type=(
            jax.ShapeDtypeStruct((BATCH, 128), jnp.float32),
            jax.ShapeDtypeStruct((BATCH, 128), jnp.float32),
        ),
        mesh=mesh,
        scratch_types=[
            pltpu.VMEM((B_PER_W,), jnp.int32),
            pltpu.VMEM((B_PER_W,), jnp.int32),
            pltpu.VMEM((B_PER_W, 128), jnp.float32),
            pltpu.SemaphoreType.DMA,
        ],
        compiler_params=pltpu.CompilerParams(needs_layout_passes=False),
    )(user_indices, item_indices, ut, it)


def _mlp_body(gu_ref, gi_ref, idxu_ref, idxi_ref, w1u_ref, w1i_ref, b1_ref,
              w2_ref, b2_ref, w3_ref, b3_ref, wo_ref, bo_ref, out_ref):
    subu = _pane_id(idxu_ref[...])
    subi = _pane_id(idxi_ref[...])
    uv = jnp.zeros_like(gu_ref[:, :EMBED])
    iv = jnp.zeros_like(uv)
    for t in range(PACK):
        uv = uv + jnp.where(subu == t, gu_ref[:, t * EMBED:(t + 1) * EMBED], 0.0)
        iv = iv + jnp.where(subi == t, gi_ref[:, t * EMBED:(t + 1) * EMBED], 0.0)
    h = jnp.dot(uv, w1u_ref[...], preferred_element_type=jnp.float32)
    h = h + jnp.dot(iv, w1i_ref[...], preferred_element_type=jnp.float32)
    h = jnp.maximum(h + b1_ref[...], 0.0)
    h = jnp.maximum(
        jnp.dot(h, w2_ref[...], preferred_element_type=jnp.float32) + b2_ref[...], 0.0)
    h = jnp.maximum(
        jnp.dot(h, w3_ref[...], preferred_element_type=jnp.float32) + b3_ref[...], 0.0)
    o = jnp.dot(h, wo_ref[...], preferred_element_type=jnp.float32) + bo_ref[...]
    out_ref[...] = jax.nn.sigmoid(o)


def _tc_mlp(gu, gi, idxu2, idxi2, W1, b1, W2, b2, W3, b3, Wo, bo):
    BB = 2048
    grid = (BATCH // BB,)
    w1u = W1[:EMBED]
    w1i = W1[EMBED:]
    full = lambda i: (0, 0)
    return pl.pallas_call(
        _mlp_body,
        grid=grid,
        in_specs=[
            pl.BlockSpec((BB, 128), lambda i: (i, 0)),
            pl.BlockSpec((BB, 128), lambda i: (i, 0)),
            pl.BlockSpec((BB, 1), lambda i: (i, 0)),
            pl.BlockSpec((BB, 1), lambda i: (i, 0)),
            pl.BlockSpec((EMBED, 128), full),
            pl.BlockSpec((EMBED, 128), full),
            pl.BlockSpec((1, 128), full),
            pl.BlockSpec((128, 64), full),
            pl.BlockSpec((1, 64), full),
            pl.BlockSpec((64, 32), full),
            pl.BlockSpec((1, 32), full),
            pl.BlockSpec((32, 1), full),
            pl.BlockSpec((1, 1), full),
        ],
        out_specs=pl.BlockSpec((BB, 1), lambda i: (i, 0)),
        out_shape=jax.ShapeDtypeStruct((BATCH, 1), jnp.float32),
    )(gu, gi, idxu2, idxi2, w1u, w1i, b1.reshape(1, 128), W2, b2.reshape(1, 64),
      W3, b3.reshape(1, 32), Wo, bo.reshape(1, 1))


def kernel(user_indices, item_indices, user_table, item_table,
           W1, b1, W2, b2, W3, b3, Wo, bo):
    uidx = user_indices.astype(jnp.int32)
    iidx = item_indices.astype(jnp.int32)
    ut = _tc_repack(user_table.T)
    it = _tc_repack(item_table.T)
    gu, gi = _sc_gather(uidx, iidx, ut, it)
    return _tc_mlp(gu, gi, uidx.reshape(BATCH, 1), iidx.reshape(BATCH, 1),
                   W1, b1, W2, b2, W3, b3, Wo, bo)


# speedup vs baseline: 2.8077x; 2.6774x over previous
import jax
import jax.numpy as jnp
from jax import lax
from jax.experimental import pallas as pl
from jax.experimental.pallas import tpu as pltpu
import jax.experimental.pallas.tpu_sc as plsc

BATCH = 16384
EMBED = 32
NROWS = 1000000
PACK = 4
C = 251904
RB = 2048
NBLK = C // RB
LAST_BLK = (NROWS + RB - 1) // RB - 1
NC = 2
NS = 16
NW = NC * NS
B_PER_W = BATCH // NW
CH = 128


def _repack_body(x0_ref, x1_ref, x2_ref, x3_ref, out_ref):
    n = EMBED * PACK
    ii = lax.broadcasted_iota(jnp.int32, (n, n), 0)
    jj = lax.broadcasted_iota(jnp.int32, (n, n), 1)
    eye = (ii == jj).astype(jnp.float32)
    xcat = jnp.concatenate(
        [x0_ref[...], x1_ref[...], x2_ref[...], x3_ref[...]], axis=0)
    out_ref[...] = lax.dot_general(
        xcat, eye, (((0,), (0,)), ((), ())),
        preferred_element_type=jnp.float32)


def _tc_repack(tabT):
    in_specs = [
        pl.BlockSpec((EMBED, RB),
                     (lambda i, t=t: (0, jnp.minimum(t * NBLK + i, LAST_BLK))))
        for t in range(PACK)
    ]
    return pl.pallas_call(
        _repack_body,
        grid=(NBLK,),
        in_specs=in_specs,
        out_specs=pl.BlockSpec((RB, EMBED * PACK), lambda i: (i, 0)),
        out_shape=jax.ShapeDtypeStruct((C, EMBED * PACK), jnp.float32),
        compiler_params=pltpu.CompilerParams(fuse_transposed_lhs_in_matmul=True),
    )(tabT, tabT, tabT, tabT)


def _pane_id(iv):
    t = (iv >= C).astype(jnp.int32)
    t = t + (iv >= 2 * C).astype(jnp.int32)
    t = t + (iv >= 3 * C).astype(jnp.int32)
    return t


def _gather_body(uidx_hbm, iidx_hbm, ut_hbm, it_hbm, gu_hbm, gi_hbm,
                 idx_v, bidx_v, g_v, sem):
    w = lax.axis_index("s") * NC + lax.axis_index("c")
    base = pl.multiple_of(w * B_PER_W, B_PER_W)

    for idx_hbm, t_hbm, o_hbm in ((uidx_hbm, ut_hbm, gu_hbm),
                                  (iidx_hbm, it_hbm, gi_hbm)):
        pltpu.sync_copy(idx_hbm.at[pl.ds(base, B_PER_W)], idx_v)
        for g in range(B_PER_W // 16):
            iv = idx_v[pl.ds(g * 16, 16)]
            bidx_v[pl.ds(g * 16, 16)] = iv - _pane_id(iv) * C
        copies = []
        for c in range(B_PER_W // CH):
            copies.append(pltpu.async_copy(
                t_hbm.at[bidx_v.at[pl.ds(c * CH, CH)]],
                g_v.at[pl.ds(c * CH, CH)], sem))
        for cp in copies:
            cp.wait()
        pltpu.sync_copy(g_v, o_hbm.at[pl.ds(base, B_PER_W)])


def _sc_gather(user_indices, item_indices, ut, it):
    mesh = plsc.VectorSubcoreMesh(core_axis_name="c", subcore_axis_name="s")
    return pl.kernel(
        _gather_body,
        out_type=(
            jax.ShapeDtypeStruct((BATCH, 128), jnp.float32),
            jax.ShapeDtypeStruct((BATCH, 128), jnp.float32),
        ),
        mesh=mesh,
        scratch_types=[
            pltpu.VMEM((B_PER_W,), jnp.int32),
            pltpu.VMEM((B_PER_W,), jnp.int32),
            pltpu.VMEM((B_PER_W, 128), jnp.float32),
            pltpu.SemaphoreType.DMA,
        ],
        compiler_params=pltpu.CompilerParams(needs_layout_passes=False),
    )(user_indices, item_indices, ut, it)


def _mlp_body(gu_ref, gi_ref, idxu_ref, idxi_ref, w1u_ref, w1i_ref, b1_ref,
              w2_ref, b2_ref, w3_ref, b3_ref, wo_ref, bo_ref, out_ref):
    subu = _pane_id(idxu_ref[...])
    subi = _pane_id(idxi_ref[...])
    lane_pane = lax.broadcasted_iota(jnp.int32, (1, PACK * EMBED), 1) // EMBED
    lr = lax.broadcasted_iota(jnp.int32, (PACK * EMBED, EMBED), 0)
    cc = lax.broadcasted_iota(jnp.int32, (PACK * EMBED, EMBED), 1)
    sel = (lr % EMBED == cc).astype(jnp.float32)
    uv = jnp.dot(jnp.where(subu == lane_pane, gu_ref[...], 0.0), sel,
                 preferred_element_type=jnp.float32)
    iv = jnp.dot(jnp.where(subi == lane_pane, gi_ref[...], 0.0), sel,
                 preferred_element_type=jnp.float32)
    h = jnp.dot(uv, w1u_ref[...], preferred_element_type=jnp.float32)
    h = h + jnp.dot(iv, w1i_ref[...], preferred_element_type=jnp.float32)
    h = jnp.maximum(h + b1_ref[...], 0.0)
    h = jnp.maximum(
        jnp.dot(h, w2_ref[...], preferred_element_type=jnp.float32) + b2_ref[...], 0.0)
    h = jnp.maximum(
        jnp.dot(h, w3_ref[...], preferred_element_type=jnp.float32) + b3_ref[...], 0.0)
    o = jnp.dot(h, wo_ref[...], preferred_element_type=jnp.float32) + bo_ref[...]
    out_ref[...] = jax.nn.sigmoid(o)


def _tc_mlp(gu, gi, idxu2, idxi2, W1, b1, W2, b2, W3, b3, Wo, bo):
    BB = 2048
    grid = (BATCH // BB,)
    w1u = W1[:EMBED]
    w1i = W1[EMBED:]
    full = lambda i: (0, 0)
    return pl.pallas_call(
        _mlp_body,
        grid=grid,
        in_specs=[
            pl.BlockSpec((BB, 128), lambda i: (i, 0)),
            pl.BlockSpec((BB, 128), lambda i: (i, 0)),
            pl.BlockSpec((BB, 1), lambda i: (i, 0)),
            pl.BlockSpec((BB, 1), lambda i: (i, 0)),
            pl.BlockSpec((EMBED, 128), full),
            pl.BlockSpec((EMBED, 128), full),
            pl.BlockSpec((1, 128), full),
            pl.BlockSpec((128, 64), full),
            pl.BlockSpec((1, 64), full),
            pl.BlockSpec((64, 32), full),
            pl.BlockSpec((1, 32), full),
            pl.BlockSpec((32, 1), full),
            pl.BlockSpec((1, 1), full),
        ],
        out_specs=pl.BlockSpec((BB, 1), lambda i: (i, 0)),
        out_shape=jax.ShapeDtypeStruct((BATCH, 1), jnp.float32),
    )(gu, gi, idxu2, idxi2, w1u, w1i, b1.reshape(1, 128), W2, b2.reshape(1, 64),
      W3, b3.reshape(1, 32), Wo, bo.reshape(1, 1))


def kernel(user_indices, item_indices, user_table, item_table,
           W1, b1, W2, b2, W3, b3, Wo, bo):
    uidx = user_indices.astype(jnp.int32)
    iidx = item_indices.astype(jnp.int32)
    ut = _tc_repack(user_table.T)
    it = _tc_repack(item_table.T)
    gu, gi = _sc_gather(uidx, iidx, ut, it)
    return _tc_mlp(gu, gi, uidx.reshape(BATCH, 1), iidx.reshape(BATCH, 1),
                   W1, b1, W2, b2, W3, b3, Wo, bo)


# speedup vs baseline: 3.6640x; 1.3050x over previous
import jax
import jax.numpy as jnp
from jax import lax
from jax.experimental import pallas as pl
from jax.experimental.pallas import tpu as pltpu
import jax.experimental.pallas.tpu_sc as plsc

BATCH = 16384
EMBED = 32
NROWS = 1000000
PACK = 4
C = 253952
RB = 4096
NBLK = C // RB
LAST_BLK = (NROWS + RB - 1) // RB - 1
NC = 2
NS = 16
NW = NC * NS
B_PER_W = BATCH // NW
CH = 128


def _repack_body(x0_ref, x1_ref, x2_ref, x3_ref, out_ref):
    n = EMBED * PACK
    ii = lax.broadcasted_iota(jnp.int32, (n, n), 0)
    jj = lax.broadcasted_iota(jnp.int32, (n, n), 1)
    eye = (ii == jj).astype(jnp.float32)
    xcat = jnp.concatenate(
        [x0_ref[...], x1_ref[...], x2_ref[...], x3_ref[...]], axis=0)
    out_ref[...] = lax.dot_general(
        xcat, eye, (((0,), (0,)), ((), ())),
        preferred_element_type=jnp.float32)


def _tc_repack(tabT):
    in_specs = [
        pl.BlockSpec((EMBED, RB),
                     (lambda i, t=t: (0, jnp.minimum(t * NBLK + i, LAST_BLK))))
        for t in range(PACK)
    ]
    return pl.pallas_call(
        _repack_body,
        grid=(NBLK,),
        in_specs=in_specs,
        out_specs=pl.BlockSpec((RB, EMBED * PACK), lambda i: (i, 0)),
        out_shape=jax.ShapeDtypeStruct((C, EMBED * PACK), jnp.float32),
        compiler_params=pltpu.CompilerParams(fuse_transposed_lhs_in_matmul=True),
    )(tabT, tabT, tabT, tabT)


def _pane_id(iv):
    t = (iv >= C).astype(jnp.int32)
    t = t + (iv >= 2 * C).astype(jnp.int32)
    t = t + (iv >= 3 * C).astype(jnp.int32)
    return t


def _gather_body(idx_hbm, t_hbm, o_hbm, idx_v, bidx_v, g_v, sem):
    w = lax.axis_index("s") * NC + lax.axis_index("c")
    base = pl.multiple_of(w * B_PER_W, B_PER_W)
    pltpu.sync_copy(idx_hbm.at[pl.ds(base, B_PER_W)], idx_v)
    for g in range(B_PER_W // 16):
        iv = idx_v[pl.ds(g * 16, 16)]
        bidx_v[pl.ds(g * 16, 16)] = iv - _pane_id(iv) * C
    copies = []
    for c in range(B_PER_W // CH):
        copies.append(pltpu.async_copy(
            t_hbm.at[bidx_v.at[pl.ds(c * CH, CH)]],
            g_v.at[pl.ds(c * CH, CH)], sem))
    for cp in copies:
        cp.wait()
    pltpu.sync_copy(g_v, o_hbm.at[pl.ds(base, B_PER_W)])


def _sc_gather(indices, tab):
    mesh = plsc.VectorSubcoreMesh(core_axis_name="c", subcore_axis_name="s")
    return pl.kernel(
        _gather_body,
        out_type=jax.ShapeDtypeStruct((BATCH, 128), jnp.float32),
        mesh=mesh,
        scratch_types=[
            pltpu.VMEM((B_PER_W,), jnp.int32),
            pltpu.VMEM((B_PER_W,), jnp.int32),
            pltpu.VMEM((B_PER_W, 128), jnp.float32),
            pltpu.SemaphoreType.DMA,
        ],
        compiler_params=pltpu.CompilerParams(needs_layout_passes=False),
    )(indices, tab)


def _mlp_body(gu_ref, gi_ref, idxu_ref, idxi_ref, w1u_ref, w1i_ref, b1_ref,
              w2_ref, b2_ref, w3_ref, b3_ref, wo_ref, bo_ref, out_ref):
    subu = _pane_id(idxu_ref[...])
    subi = _pane_id(idxi_ref[...])
    lane_pane = lax.broadcasted_iota(jnp.int32, (1, PACK * EMBED), 1) // EMBED
    lr = lax.broadcasted_iota(jnp.int32, (PACK * EMBED, EMBED), 0)
    cc = lax.broadcasted_iota(jnp.int32, (PACK * EMBED, EMBED), 1)
    sel = (lr % EMBED == cc).astype(jnp.float32)
    uv = jnp.dot(jnp.where(subu == lane_pane, gu_ref[...], 0.0), sel,
                 preferred_element_type=jnp.float32)
    iv = jnp.dot(jnp.where(subi == lane_pane, gi_ref[...], 0.0), sel,
                 preferred_element_type=jnp.float32)
    h = jnp.dot(uv, w1u_ref[...], preferred_element_type=jnp.float32)
    h = h + jnp.dot(iv, w1i_ref[...], preferred_element_type=jnp.float32)
    h = jnp.maximum(h + b1_ref[...], 0.0)
    h = jnp.maximum(
        jnp.dot(h, w2_ref[...], preferred_element_type=jnp.float32) + b2_ref[...], 0.0)
    h = jnp.maximum(
        jnp.dot(h, w3_ref[...], preferred_element_type=jnp.float32) + b3_ref[...], 0.0)
    o = jnp.dot(h, wo_ref[...], preferred_element_type=jnp.float32) + bo_ref[...]
    out_ref[...] = jax.nn.sigmoid(o)


def _tc_mlp(gu, gi, idxu2, idxi2, W1, b1, W2, b2, W3, b3, Wo, bo):
    BB = 2048
    grid = (BATCH // BB,)
    w1u = W1[:EMBED]
    w1i = W1[EMBED:]
    full = lambda i: (0, 0)
    return pl.pallas_call(
        _mlp_body,
        grid=grid,
        in_specs=[
            pl.BlockSpec((BB, 128), lambda i: (i, 0)),
            pl.BlockSpec((BB, 128), lambda i: (i, 0)),
            pl.BlockSpec((BB, 1), lambda i: (i, 0)),
            pl.BlockSpec((BB, 1), lambda i: (i, 0)),
            pl.BlockSpec((EMBED, 128), full),
            pl.BlockSpec((EMBED, 128), full),
            pl.BlockSpec((1, 128), full),
            pl.BlockSpec((128, 64), full),
            pl.BlockSpec((1, 64), full),
            pl.BlockSpec((64, 32), full),
            pl.BlockSpec((1, 32), full),
            pl.BlockSpec((32, 1), full),
            pl.BlockSpec((1, 1), full),
        ],
        out_specs=pl.BlockSpec((BB, 1), lambda i: (i, 0)),
        out_shape=jax.ShapeDtypeStruct((BATCH, 1), jnp.float32),
    )(gu, gi, idxu2, idxi2, w1u, w1i, b1.reshape(1, 128), W2, b2.reshape(1, 64),
      W3, b3.reshape(1, 32), Wo, bo.reshape(1, 1))


def kernel(user_indices, item_indices, user_table, item_table,
           W1, b1, W2, b2, W3, b3, Wo, bo):
    uidx = user_indices.astype(jnp.int32)
    iidx = item_indices.astype(jnp.int32)
    ut = _tc_repack(user_table.T)
    gu = _sc_gather(uidx, ut)
    it = _tc_repack(item_table.T)
    gi = _sc_gather(iidx, it)
    return _tc_mlp(gu, gi, uidx.reshape(BATCH, 1), iidx.reshape(BATCH, 1),
                   W1, b1, W2, b2, W3, b3, Wo, bo)


# speedup vs baseline: 3.8927x; 1.0624x over previous
import jax
import jax.numpy as jnp
from jax import lax
from jax.experimental import pallas as pl
from jax.experimental.pallas import tpu as pltpu
import jax.experimental.pallas.tpu_sc as plsc

BATCH = 16384
EMBED = 32
NROWS = 1000000
PACK = 4
C = 253952
RB = 4096
NBLK = C // RB
LAST_BLK = (NROWS + RB - 1) // RB - 1
NC = 2
NS = 16
NW = NC * NS
B_PER_W = BATCH // NW
CH = 128


def _repack_body(x0_ref, x1_ref, x2_ref, x3_ref, out_ref):
    n = EMBED * PACK
    i = pl.program_id(0)
    ii = lax.broadcasted_iota(jnp.int32, (n, n), 0)
    jj = lax.broadcasted_iota(jnp.int32, (n, n), 1)
    eye = (ii == jj).astype(jnp.float32)
    lane = lax.broadcasted_iota(jnp.int32, (EMBED, RB), 1)
    parts = []
    for t, r in enumerate((x0_ref, x1_ref, x2_ref, x3_ref)):
        ub = (t * NBLK + i) * RB
        parts.append(jnp.where(lane + ub < NROWS, r[...], 0.0))
    xcat = jnp.concatenate(parts, axis=0)
    out_ref[...] = lax.dot_general(
        xcat, eye, (((0,), (0,)), ((), ())),
        preferred_element_type=jnp.float32)


def _tc_repack(tabT):
    in_specs = [
        pl.BlockSpec((EMBED, RB),
                     (lambda i, t=t: (0, jnp.minimum(t * NBLK + i, LAST_BLK))))
        for t in range(PACK)
    ]
    return pl.pallas_call(
        _repack_body,
        grid=(NBLK,),
        in_specs=in_specs,
        out_specs=pl.BlockSpec((RB, EMBED * PACK), lambda i: (i, 0)),
        out_shape=jax.ShapeDtypeStruct((C, EMBED * PACK), jnp.float32),
        compiler_params=pltpu.CompilerParams(fuse_transposed_lhs_in_matmul=True),
    )(tabT, tabT, tabT, tabT)


def _pane_id(iv):
    t = (iv >= C).astype(jnp.int32)
    t = t + (iv >= 2 * C).astype(jnp.int32)
    t = t + (iv >= 3 * C).astype(jnp.int32)
    return t


def _gather_body(idx_hbm, t_hbm, o_hbm, idx_v, bidx_v, g_v, sem):
    w = lax.axis_index("s") * NC + lax.axis_index("c")
    base = pl.multiple_of(w * B_PER_W, B_PER_W)
    pltpu.sync_copy(idx_hbm.at[pl.ds(base, B_PER_W)], idx_v)
    for g in range(B_PER_W // 16):
        iv = idx_v[pl.ds(g * 16, 16)]
        bidx_v[pl.ds(g * 16, 16)] = iv - _pane_id(iv) * C
    copies = []
    for c in range(B_PER_W // CH):
        copies.append(pltpu.async_copy(
            t_hbm.at[bidx_v.at[pl.ds(c * CH, CH)]],
            g_v.at[pl.ds(c * CH, CH)], sem))
    for cp in copies:
        cp.wait()
    pltpu.sync_copy(g_v, o_hbm.at[pl.ds(base, B_PER_W)])


def _sc_gather(indices, tab):
    mesh = plsc.VectorSubcoreMesh(core_axis_name="c", subcore_axis_name="s")
    return pl.kernel(
        _gather_body,
        out_type=jax.ShapeDtypeStruct((BATCH, 128), jnp.float32),
        mesh=mesh,
        scratch_types=[
            pltpu.VMEM((B_PER_W,), jnp.int32),
            pltpu.VMEM((B_PER_W,), jnp.int32),
            pltpu.VMEM((B_PER_W, 128), jnp.float32),
            pltpu.SemaphoreType.DMA,
        ],
        compiler_params=pltpu.CompilerParams(needs_layout_passes=False),
    )(indices, tab)


def _mlp_body(gu_ref, gi_ref, ou_ref, oi_ref, w1u_ref, w1i_ref, b1_ref,
              w2_ref, b2_ref, w3_ref, b3_ref, wo_ref, bo_ref, out_ref):
    tt = lax.broadcasted_iota(jnp.int32, (PACK, PACK * EMBED), 0)
    ll = lax.broadcasted_iota(jnp.int32, (PACK, PACK * EMBED), 1) // EMBED
    pmat = (tt == ll).astype(jnp.float32)
    lr = lax.broadcasted_iota(jnp.int32, (PACK * EMBED, EMBED), 0)
    cc = lax.broadcasted_iota(jnp.int32, (PACK * EMBED, EMBED), 1)
    sel = (lr % EMBED == cc).astype(jnp.float32)
    mu = lax.dot_general(ou_ref[...], pmat, (((0,), (0,)), ((), ())),
                         preferred_element_type=jnp.float32)
    mi = lax.dot_general(oi_ref[...], pmat, (((0,), (0,)), ((), ())),
                         preferred_element_type=jnp.float32)
    uv = jnp.dot(gu_ref[...] * mu, sel, preferred_element_type=jnp.float32)
    iv = jnp.dot(gi_ref[...] * mi, sel, preferred_element_type=jnp.float32)
    h = jnp.dot(uv, w1u_ref[...], preferred_element_type=jnp.float32)
    h = h + jnp.dot(iv, w1i_ref[...], preferred_element_type=jnp.float32)
    h = jnp.maximum(h + b1_ref[...], 0.0)
    h = jnp.maximum(
        jnp.dot(h, w2_ref[...], preferred_element_type=jnp.float32) + b2_ref[...], 0.0)
    h = jnp.maximum(
        jnp.dot(h, w3_ref[...], preferred_element_type=jnp.float32) + b3_ref[...], 0.0)
    o = lax.dot_general(wo_ref[...], h, (((0,), (1,)), ((), ())),
                        preferred_element_type=jnp.float32) + bo_ref[...]
    out_ref[...] = jax.nn.sigmoid(o)


def _tc_mlp(gu, gi, ou, oi, W1, b1, W2, b2, W3, b3, Wo, bo):
    BB = 2048
    grid = (BATCH // BB,)
    w1u = W1[:EMBED]
    w1i = W1[EMBED:]
    full = lambda i: (0, 0)
    return pl.pallas_call(
        _mlp_body,
        grid=grid,
        in_specs=[
            pl.BlockSpec((BB, 128), lambda i: (i, 0)),
            pl.BlockSpec((BB, 128), lambda i: (i, 0)),
            pl.BlockSpec((PACK, BB), lambda i: (0, i)),
            pl.BlockSpec((PACK, BB), lambda i: (0, i)),
            pl.BlockSpec((EMBED, 128), full),
            pl.BlockSpec((EMBED, 128), full),
            pl.BlockSpec((1, 128), full),
            pl.BlockSpec((128, 64), full),
            pl.BlockSpec((1, 64), full),
            pl.BlockSpec((64, 32), full),
            pl.BlockSpec((1, 32), full),
            pl.BlockSpec((32, 1), full),
            pl.BlockSpec((1, 1), full),
        ],
        out_specs=pl.BlockSpec((1, BB), lambda i: (0, i)),
        out_shape=jax.ShapeDtypeStruct((1, BATCH), jnp.float32),
    )(gu, gi, ou, oi, w1u, w1i, b1.reshape(1, 128), W2, b2.reshape(1, 64),
      W3, b3.reshape(1, 32), Wo, bo.reshape(1, 1))


def kernel(user_indices, item_indices, user_table, item_table,
           W1, b1, W2, b2, W3, b3, Wo, bo):
    uidx = user_indices.astype(jnp.int32)
    iidx = item_indices.astype(jnp.int32)
    panes = jnp.arange(PACK, dtype=jnp.int32)[:, None]
    ou = (_pane_id(uidx)[None, :] == panes).astype(jnp.float32)
    oi = (_pane_id(iidx)[None, :] == panes).astype(jnp.float32)
    ut = _tc_repack(user_table.T)
    gu = _sc_gather(uidx, ut)
    it = _tc_repack(item_table.T)
    gi = _sc_gather(iidx, it)
    oT = _tc_mlp(gu, gi, ou, oi, W1, b1, W2, b2, W3, b3, Wo, bo)
    return oT.reshape(BATCH, 1)


# speedup vs baseline: 4.4114x; 1.1332x over previous
import jax
import jax.numpy as jnp
from jax import lax
from jax.experimental import pallas as pl
from jax.experimental.pallas import tpu as pltpu
import jax.experimental.pallas.tpu_sc as plsc

BATCH = 16384
EMBED = 32
NROWS = 1000000
PACK = 4
C = 253952
RB = 8192
NBLK = C // RB
LAST_BLK = (NROWS + RB - 1) // RB - 1
NC = 2
NS = 16
NW = NC * NS
B_PER_W = BATCH // NW
CH = 128


def _repack_body(x0_ref, x1_ref, x2_ref, x3_ref, out_ref):
    n = EMBED * PACK
    i = pl.program_id(0)
    ii = lax.broadcasted_iota(jnp.int32, (n, n), 0)
    jj = lax.broadcasted_iota(jnp.int32, (n, n), 1)
    eye = (ii == jj).astype(jnp.float32)
    lane = lax.broadcasted_iota(jnp.int32, (EMBED, RB), 1)
    parts = []
    for t, r in enumerate((x0_ref, x1_ref, x2_ref, x3_ref)):
        ub = (t * NBLK + i) * RB
        parts.append(jnp.where(lane + ub < NROWS, r[...], 0.0))
    xcat = jnp.concatenate(parts, axis=0)
    out_ref[...] = lax.dot_general(
        xcat, eye, (((0,), (0,)), ((), ())),
        preferred_element_type=jnp.float32)


def _tc_repack(tabT):
    in_specs = [
        pl.BlockSpec((EMBED, RB),
                     (lambda i, t=t: (0, jnp.minimum(t * NBLK + i, LAST_BLK))))
        for t in range(PACK)
    ]
    return pl.pallas_call(
        _repack_body,
        grid=(NBLK,),
        in_specs=in_specs,
        out_specs=pl.BlockSpec((RB, EMBED * PACK), lambda i: (i, 0)),
        out_shape=jax.ShapeDtypeStruct((C, EMBED * PACK), jnp.float32),
        compiler_params=pltpu.CompilerParams(fuse_transposed_lhs_in_matmul=True),
    )(tabT, tabT, tabT, tabT)


def _pane_id(iv):
    t = (iv >= C).astype(jnp.int32)
    t = t + (iv >= 2 * C).astype(jnp.int32)
    t = t + (iv >= 3 * C).astype(jnp.int32)
    return t


def _gather_body(idx_hbm, t_hbm, o_hbm, idx_v, bidx_v, g_v, sem):
    w = lax.axis_index("s") * NC + lax.axis_index("c")
    base = pl.multiple_of(w * B_PER_W, B_PER_W)
    pltpu.sync_copy(idx_hbm.at[pl.ds(base, B_PER_W)], idx_v)
    for g in range(B_PER_W // 16):
        iv = idx_v[pl.ds(g * 16, 16)]
        bidx_v[pl.ds(g * 16, 16)] = iv - _pane_id(iv) * C
    copies = []
    for c in range(B_PER_W // CH):
        copies.append(pltpu.async_copy(
            t_hbm.at[bidx_v.at[pl.ds(c * CH, CH)]],
            g_v.at[pl.ds(c * CH, CH)], sem))
    for cp in copies:
        cp.wait()
    pltpu.sync_copy(g_v, o_hbm.at[pl.ds(base, B_PER_W)])


def _sc_gather(indices, tab):
    mesh = plsc.VectorSubcoreMesh(core_axis_name="c", subcore_axis_name="s")
    return pl.kernel(
        _gather_body,
        out_type=jax.ShapeDtypeStruct((BATCH, 128), jnp.float32),
        mesh=mesh,
        scratch_types=[
            pltpu.VMEM((B_PER_W,), jnp.int32),
            pltpu.VMEM((B_PER_W,), jnp.int32),
            pltpu.VMEM((B_PER_W, 128), jnp.float32),
            pltpu.SemaphoreType.DMA,
        ],
        compiler_params=pltpu.CompilerParams(needs_layout_passes=False),
    )(indices, tab)


def _mlp_body(gu_ref, gi_ref, ou_ref, oi_ref, w1u_ref, w1i_ref, b1_ref,
              w2_ref, b2_ref, w3_ref, b3_ref, wo_ref, bo_ref, out_ref):
    tt = lax.broadcasted_iota(jnp.int32, (PACK, PACK * EMBED), 0)
    ll = lax.broadcasted_iota(jnp.int32, (PACK, PACK * EMBED), 1) // EMBED
    pmat = (tt == ll).astype(jnp.float32)
    lr = lax.broadcasted_iota(jnp.int32, (PACK * EMBED, EMBED), 0)
    cc = lax.broadcasted_iota(jnp.int32, (PACK * EMBED, EMBED), 1)
    sel = (lr % EMBED == cc).astype(jnp.float32)
    mu = lax.dot_general(ou_ref[...], pmat, (((0,), (0,)), ((), ())),
                         preferred_element_type=jnp.float32)
    mi = lax.dot_general(oi_ref[...], pmat, (((0,), (0,)), ((), ())),
                         preferred_element_type=jnp.float32)
    uv = jnp.dot(gu_ref[...] * mu, sel, preferred_element_type=jnp.float32)
    iv = jnp.dot(gi_ref[...] * mi, sel, preferred_element_type=jnp.float32)
    h = jnp.dot(uv, w1u_ref[...], preferred_element_type=jnp.float32)
    h = h + jnp.dot(iv, w1i_ref[...], preferred_element_type=jnp.float32)
    h = jnp.maximum(h + b1_ref[...], 0.0)
    h = jnp.maximum(
        jnp.dot(h, w2_ref[...], preferred_element_type=jnp.float32) + b2_ref[...], 0.0)
    h = jnp.maximum(
        jnp.dot(h, w3_ref[...], preferred_element_type=jnp.float32) + b3_ref[...], 0.0)
    o = lax.dot_general(wo_ref[...], h, (((0,), (1,)), ((), ())),
                        preferred_element_type=jnp.float32) + bo_ref[...]
    out_ref[...] = jax.nn.sigmoid(o)


def _tc_mlp(gu, gi, ou, oi, W1, b1, W2, b2, W3, b3, Wo, bo):
    BB = 2048
    grid = (BATCH // BB,)
    w1u = W1[:EMBED]
    w1i = W1[EMBED:]
    full = lambda i: (0, 0)
    return pl.pallas_call(
        _mlp_body,
        grid=grid,
        in_specs=[
            pl.BlockSpec((BB, 128), lambda i: (i, 0)),
            pl.BlockSpec((BB, 128), lambda i: (i, 0)),
            pl.BlockSpec((PACK, BB), lambda i: (0, i)),
            pl.BlockSpec((PACK, BB), lambda i: (0, i)),
            pl.BlockSpec((EMBED, 128), full),
            pl.BlockSpec((EMBED, 128), full),
            pl.BlockSpec((1, 128), full),
            pl.BlockSpec((128, 64), full),
            pl.BlockSpec((1, 64), full),
            pl.BlockSpec((64, 32), full),
            pl.BlockSpec((1, 32), full),
            pl.BlockSpec((32, 1), full),
            pl.BlockSpec((1, 1), full),
        ],
        out_specs=pl.BlockSpec((1, BB), lambda i: (0, i)),
        out_shape=jax.ShapeDtypeStruct((1, BATCH), jnp.float32),
    )(gu, gi, ou, oi, w1u, w1i, b1.reshape(1, 128), W2, b2.reshape(1, 64),
      W3, b3.reshape(1, 32), Wo, bo.reshape(1, 1))


def kernel(user_indices, item_indices, user_table, item_table,
           W1, b1, W2, b2, W3, b3, Wo, bo):
    uidx = user_indices.astype(jnp.int32)
    iidx = item_indices.astype(jnp.int32)
    panes = jnp.arange(PACK, dtype=jnp.int32)[:, None]
    ou = (_pane_id(uidx)[None, :] == panes).astype(jnp.float32)
    oi = (_pane_id(iidx)[None, :] == panes).astype(jnp.float32)
    ut = _tc_repack(user_table.T)
    gu = _sc_gather(uidx, ut)
    it = _tc_repack(item_table.T)
    gi = _sc_gather(iidx, it)
    oT = _tc_mlp(gu, gi, ou, oi, W1, b1, W2, b2, W3, b3, Wo, bo)
    return oT.reshape(BATCH, 1)


# speedup vs baseline: 4.4501x; 1.0088x over previous
import jax
import jax.numpy as jnp
from jax import lax
from jax.experimental import pallas as pl
from jax.experimental.pallas import tpu as pltpu
import jax.experimental.pallas.tpu_sc as plsc

BATCH = 16384
EMBED = 32
NROWS = 1000000
PACK = 4
C = 262144
RB = 16384
NBLK = C // RB
LAST_BLK = (NROWS + RB - 1) // RB - 1
NC = 2
NS = 16
NW = NC * NS
B_PER_W = BATCH // NW
CH = 128


def _repack_body(x0_ref, x1_ref, x2_ref, x3_ref, out_ref):
    n = EMBED * PACK
    i = pl.program_id(0)
    ii = lax.broadcasted_iota(jnp.int32, (n, n), 0)
    jj = lax.broadcasted_iota(jnp.int32, (n, n), 1)
    eye = (ii == jj).astype(jnp.float32)
    lane = lax.broadcasted_iota(jnp.int32, (EMBED, RB), 1)
    parts = []
    for t, r in enumerate((x0_ref, x1_ref, x2_ref, x3_ref)):
        ub = (t * NBLK + i) * RB
        parts.append(jnp.where(lane + ub < NROWS, r[...], 0.0))
    xcat = jnp.concatenate(parts, axis=0)
    out_ref[...] = lax.dot_general(
        xcat, eye, (((0,), (0,)), ((), ())),
        preferred_element_type=jnp.float32)


def _tc_repack(tabT):
    in_specs = [
        pl.BlockSpec((EMBED, RB),
                     (lambda i, t=t: (0, jnp.minimum(t * NBLK + i, LAST_BLK))))
        for t in range(PACK)
    ]
    return pl.pallas_call(
        _repack_body,
        grid=(NBLK,),
        in_specs=in_specs,
        out_specs=pl.BlockSpec((RB, EMBED * PACK), lambda i: (i, 0)),
        out_shape=jax.ShapeDtypeStruct((C, EMBED * PACK), jnp.float32),
        compiler_params=pltpu.CompilerParams(fuse_transposed_lhs_in_matmul=True),
    )(tabT, tabT, tabT, tabT)


def _pane_id(iv):
    t = (iv >= C).astype(jnp.int32)
    t = t + (iv >= 2 * C).astype(jnp.int32)
    t = t + (iv >= 3 * C).astype(jnp.int32)
    return t


def _gather_body(idx_hbm, t_hbm, o_hbm, idx_v, bidx_v, g_v, sem):
    w = lax.axis_index("s") * NC + lax.axis_index("c")
    base = pl.multiple_of(w * B_PER_W, B_PER_W)
    pltpu.sync_copy(idx_hbm.at[pl.ds(base, B_PER_W)], idx_v)
    for g in range(B_PER_W // 16):
        iv = idx_v[pl.ds(g * 16, 16)]
        bidx_v[pl.ds(g * 16, 16)] = iv - _pane_id(iv) * C
    copies = []
    for c in range(B_PER_W // CH):
        copies.append(pltpu.async_copy(
            t_hbm.at[bidx_v.at[pl.ds(c * CH, CH)]],
            g_v.at[pl.ds(c * CH, CH)], sem))
    for cp in copies:
        cp.wait()
    pltpu.sync_copy(g_v, o_hbm.at[pl.ds(base, B_PER_W)])


def _sc_gather(indices, tab):
    mesh = plsc.VectorSubcoreMesh(core_axis_name="c", subcore_axis_name="s")
    return pl.kernel(
        _gather_body,
        out_type=jax.ShapeDtypeStruct((BATCH, 128), jnp.float32),
        mesh=mesh,
        scratch_types=[
            pltpu.VMEM((B_PER_W,), jnp.int32),
            pltpu.VMEM((B_PER_W,), jnp.int32),
            pltpu.VMEM((B_PER_W, 128), jnp.float32),
            pltpu.SemaphoreType.DMA,
        ],
        compiler_params=pltpu.CompilerParams(needs_layout_passes=False),
    )(indices, tab)


def _mlp_body(gu_ref, gi_ref, ou_ref, oi_ref, w1u_ref, w1i_ref, b1_ref,
              w2_ref, b2_ref, w3_ref, b3_ref, wo_ref, bo_ref, out_ref):
    tt = lax.broadcasted_iota(jnp.int32, (PACK, PACK * EMBED), 0)
    ll = lax.broadcasted_iota(jnp.int32, (PACK, PACK * EMBED), 1) // EMBED
    pmat = (tt == ll).astype(jnp.float32)
    lr = lax.broadcasted_iota(jnp.int32, (PACK * EMBED, EMBED), 0)
    cc = lax.broadcasted_iota(jnp.int32, (PACK * EMBED, EMBED), 1)
    sel = (lr % EMBED == cc).astype(jnp.float32)
    mu = lax.dot_general(ou_ref[...], pmat, (((0,), (0,)), ((), ())),
                         preferred_element_type=jnp.float32)
    mi = lax.dot_general(oi_ref[...], pmat, (((0,), (0,)), ((), ())),
                         preferred_element_type=jnp.float32)
    uv = jnp.dot(gu_ref[...] * mu, sel, preferred_element_type=jnp.float32)
    iv = jnp.dot(gi_ref[...] * mi, sel, preferred_element_type=jnp.float32)
    h = jnp.dot(uv, w1u_ref[...], preferred_element_type=jnp.float32)
    h = h + jnp.dot(iv, w1i_ref[...], preferred_element_type=jnp.float32)
    h = jnp.maximum(h + b1_ref[...], 0.0)
    h = jnp.maximum(
        jnp.dot(h, w2_ref[...], preferred_element_type=jnp.float32) + b2_ref[...], 0.0)
    h = jnp.maximum(
        jnp.dot(h, w3_ref[...], preferred_element_type=jnp.float32) + b3_ref[...], 0.0)
    o = lax.dot_general(wo_ref[...], h, (((0,), (1,)), ((), ())),
                        preferred_element_type=jnp.float32) + bo_ref[...]
    out_ref[...] = jax.nn.sigmoid(o)


def _tc_mlp(gu, gi, ou, oi, W1, b1, W2, b2, W3, b3, Wo, bo):
    BB = 2048
    grid = (BATCH // BB,)
    w1u = W1[:EMBED]
    w1i = W1[EMBED:]
    full = lambda i: (0, 0)
    return pl.pallas_call(
        _mlp_body,
        grid=grid,
        in_specs=[
            pl.BlockSpec((BB, 128), lambda i: (i, 0)),
            pl.BlockSpec((BB, 128), lambda i: (i, 0)),
            pl.BlockSpec((PACK, BB), lambda i: (0, i)),
            pl.BlockSpec((PACK, BB), lambda i: (0, i)),
            pl.BlockSpec((EMBED, 128), full),
            pl.BlockSpec((EMBED, 128), full),
            pl.BlockSpec((1, 128), full),
            pl.BlockSpec((128, 64), full),
            pl.BlockSpec((1, 64), full),
            pl.BlockSpec((64, 32), full),
            pl.BlockSpec((1, 32), full),
            pl.BlockSpec((32, 1), full),
            pl.BlockSpec((1, 1), full),
        ],
        out_specs=pl.BlockSpec((1, BB), lambda i: (0, i)),
        out_shape=jax.ShapeDtypeStruct((1, BATCH), jnp.float32),
    )(gu, gi, ou, oi, w1u, w1i, b1.reshape(1, 128), W2, b2.reshape(1, 64),
      W3, b3.reshape(1, 32), Wo, bo.reshape(1, 1))


def kernel(user_indices, item_indices, user_table, item_table,
           W1, b1, W2, b2, W3, b3, Wo, bo):
    uidx = user_indices.astype(jnp.int32)
    iidx = item_indices.astype(jnp.int32)
    panes = jnp.arange(PACK, dtype=jnp.int32)[:, None]
    ou = (_pane_id(uidx)[None, :] == panes).astype(jnp.float32)
    oi = (_pane_id(iidx)[None, :] == panes).astype(jnp.float32)
    ut = _tc_repack(user_table.T)
    gu = _sc_gather(uidx, ut)
    it = _tc_repack(item_table.T)
    gi = _sc_gather(iidx, it)
    oT = _tc_mlp(gu, gi, ou, oi, W1, b1, W2, b2, W3, b3, Wo, bo)
    return oT.reshape(BATCH, 1)


# speedup vs baseline: 4.4828x; 1.0073x over previous
import jax
import jax.numpy as jnp
from jax import lax
from jax.experimental import pallas as pl
from jax.experimental.pallas import tpu as pltpu
import jax.experimental.pallas.tpu_sc as plsc

BATCH = 16384
EMBED = 32
NROWS = 1000000
PACK = 4
C = 262144
RB = 16384
NBLK = C // RB
LAST_BLK = (NROWS + RB - 1) // RB - 1
NC = 2
NS = 16
NW = NC * NS
B_PER_W = BATCH // NW
CH = 128


def _repack_body(x0_ref, x1_ref, x2_ref, x3_ref, out_ref):
    n = EMBED * PACK
    i = pl.program_id(0)
    ii = lax.broadcasted_iota(jnp.int32, (n, n), 0)
    jj = lax.broadcasted_iota(jnp.int32, (n, n), 1)
    eye = (ii == jj).astype(jnp.float32)
    lane = lax.broadcasted_iota(jnp.int32, (EMBED, RB), 1)
    parts = []
    for t, r in enumerate((x0_ref, x1_ref, x2_ref, x3_ref)):
        ub = (t * NBLK + i) * RB
        parts.append(jnp.where(lane + ub < NROWS, r[...], 0.0))
    xcat = jnp.concatenate(parts, axis=0)
    out_ref[...] = lax.dot_general(
        xcat, eye, (((0,), (0,)), ((), ())),
        preferred_element_type=jnp.float32)


def _tc_repack(tabT):
    in_specs = [
        pl.BlockSpec((EMBED, RB),
                     (lambda i, t=t: (0, jnp.minimum(t * NBLK + i, LAST_BLK))))
        for t in range(PACK)
    ]
    return pl.pallas_call(
        _repack_body,
        grid=(NBLK,),
        in_specs=in_specs,
        out_specs=pl.BlockSpec((RB, EMBED * PACK), lambda i: (i, 0)),
        out_shape=jax.ShapeDtypeStruct((C, EMBED * PACK), jnp.float32),
        compiler_params=pltpu.CompilerParams(fuse_transposed_lhs_in_matmul=True),
    )(tabT, tabT, tabT, tabT)


def _pane_id(iv):
    t = (iv >= C).astype(jnp.int32)
    t = t + (iv >= 2 * C).astype(jnp.int32)
    t = t + (iv >= 3 * C).astype(jnp.int32)
    return t


def _gather_body(idx_hbm, t_hbm, o_hbm, idx_v, bidx_v, g_v, sem):
    w = lax.axis_index("s") * NC + lax.axis_index("c")
    base = pl.multiple_of(w * B_PER_W, B_PER_W)
    pltpu.sync_copy(idx_hbm.at[pl.ds(base, B_PER_W)], idx_v)
    for g in range(B_PER_W // 16):
        iv = idx_v[pl.ds(g * 16, 16)]
        bidx_v[pl.ds(g * 16, 16)] = iv - _pane_id(iv) * C
    copies = []
    for c in range(B_PER_W // CH):
        copies.append(pltpu.async_copy(
            t_hbm.at[bidx_v.at[pl.ds(c * CH, CH)]],
            g_v.at[pl.ds(c * CH, CH)], sem))
    for cp in copies:
        cp.wait()
    pltpu.sync_copy(g_v, o_hbm.at[pl.ds(base, B_PER_W)])


def _sc_gather(indices, tab):
    mesh = plsc.VectorSubcoreMesh(core_axis_name="c", subcore_axis_name="s")
    return pl.kernel(
        _gather_body,
        out_type=jax.ShapeDtypeStruct((BATCH, 128), jnp.float32),
        mesh=mesh,
        scratch_types=[
            pltpu.VMEM((B_PER_W,), jnp.int32),
            pltpu.VMEM((B_PER_W,), jnp.int32),
            pltpu.VMEM((B_PER_W, 128), jnp.float32),
            pltpu.SemaphoreType.DMA,
        ],
        compiler_params=pltpu.CompilerParams(needs_layout_passes=False),
    )(indices, tab)


def _mlp_body(gu_ref, gi_ref, ou_ref, oi_ref, w1u_ref, w1i_ref, b1_ref,
              w2_ref, b2_ref, w3_ref, b3_ref, wo_ref, bo_ref, out_ref):
    tt = lax.broadcasted_iota(jnp.int32, (PACK, PACK * EMBED), 0)
    ll = lax.broadcasted_iota(jnp.int32, (PACK, PACK * EMBED), 1) // EMBED
    pmat = (tt == ll).astype(jnp.float32)
    lr = lax.broadcasted_iota(jnp.int32, (PACK * EMBED, EMBED), 0)
    cc = lax.broadcasted_iota(jnp.int32, (PACK * EMBED, EMBED), 1)
    sel = (lr % EMBED == cc).astype(jnp.float32)
    mu = lax.dot_general(ou_ref[...], pmat, (((0,), (0,)), ((), ())),
                         preferred_element_type=jnp.float32)
    mi = lax.dot_general(oi_ref[...], pmat, (((0,), (0,)), ((), ())),
                         preferred_element_type=jnp.float32)
    uv = jnp.dot(gu_ref[...] * mu, sel, preferred_element_type=jnp.float32)
    iv = jnp.dot(gi_ref[...] * mi, sel, preferred_element_type=jnp.float32)
    h = jnp.dot(uv, w1u_ref[...], preferred_element_type=jnp.float32)
    h = h + jnp.dot(iv, w1i_ref[...], preferred_element_type=jnp.float32)
    h = jnp.maximum(h + b1_ref[...], 0.0)
    h = jnp.maximum(
        jnp.dot(h, w2_ref[...], preferred_element_type=jnp.float32) + b2_ref[...], 0.0)
    h = jnp.maximum(
        jnp.dot(h, w3_ref[...], preferred_element_type=jnp.float32) + b3_ref[...], 0.0)
    o = lax.dot_general(wo_ref[...], h, (((0,), (1,)), ((), ())),
                        preferred_element_type=jnp.float32) + bo_ref[...]
    out_ref[...] = jax.nn.sigmoid(o)


def _tc_mlp(gu, gi, ou, oi, W1, b1, W2, b2, W3, b3, Wo, bo):
    BB = 4096
    grid = (BATCH // BB,)
    w1u = W1[:EMBED]
    w1i = W1[EMBED:]
    full = lambda i: (0, 0)
    return pl.pallas_call(
        _mlp_body,
        grid=grid,
        in_specs=[
            pl.BlockSpec((BB, 128), lambda i: (i, 0)),
            pl.BlockSpec((BB, 128), lambda i: (i, 0)),
            pl.BlockSpec((PACK, BB), lambda i: (0, i)),
            pl.BlockSpec((PACK, BB), lambda i: (0, i)),
            pl.BlockSpec((EMBED, 128), full),
            pl.BlockSpec((EMBED, 128), full),
            pl.BlockSpec((1, 128), full),
            pl.BlockSpec((128, 64), full),
            pl.BlockSpec((1, 64), full),
            pl.BlockSpec((64, 32), full),
            pl.BlockSpec((1, 32), full),
            pl.BlockSpec((32, 1), full),
            pl.BlockSpec((1, 1), full),
        ],
        out_specs=pl.BlockSpec((1, BB), lambda i: (0, i)),
        out_shape=jax.ShapeDtypeStruct((1, BATCH), jnp.float32),
    )(gu, gi, ou, oi, w1u, w1i, b1.reshape(1, 128), W2, b2.reshape(1, 64),
      W3, b3.reshape(1, 32), Wo, bo.reshape(1, 1))


def kernel(user_indices, item_indices, user_table, item_table,
           W1, b1, W2, b2, W3, b3, Wo, bo):
    uidx = user_indices.astype(jnp.int32)
    iidx = item_indices.astype(jnp.int32)
    panes = jnp.arange(PACK, dtype=jnp.int32)[:, None]
    ou = (_pane_id(uidx)[None, :] == panes).astype(jnp.float32)
    oi = (_pane_id(iidx)[None, :] == panes).astype(jnp.float32)
    ut = _tc_repack(user_table.T)
    gu = _sc_gather(uidx, ut)
    it = _tc_repack(item_table.T)
    gi = _sc_gather(iidx, it)
    oT = _tc_mlp(gu, gi, ou, oi, W1, b1, W2, b2, W3, b3, Wo, bo)
    return oT.reshape(BATCH, 1)
